# Initial kernel scaffold; baseline (speedup 1.0000x reference)
import jax
import jax.numpy as jnp
from jax.experimental import pallas as pl


def kernel(q, k_ctx, v_ctx, q_for_mem, mem_keys, mem_values):
    raise NotImplementedError("write your pallas kernel here")


# speedup vs baseline: 6.5183x; 6.5183x over previous
import math

import jax
import jax.numpy as jnp
from jax import lax
from jax.experimental import pallas as pl
from jax.experimental.pallas import tpu as pltpu
from jax.experimental.pallas import tpu_sc as plsc

_B, _H, _TQ, _DH = 8, 16, 4, 128
_TK = 2048
_N = 100000
_K = 32
_NQ = _B * _TQ

_W = 6400
_NB = 16
_NPAD = _W * _NB

_CAP = 48
_IMIN_I = -(2 ** 31)
_IMAX_I = 2 ** 31 - 1


def _score_body(qf_ref, mk_ref, s_ref):
    s = lax.dot_general(
        qf_ref[...], mk_ref[...], (((1,), (1,)), ((), ())),
        preferred_element_type=jnp.float32)
    b = jax.lax.bitcast_convert_type(s, jnp.int32)
    s_ref[...] = jnp.where(b >= 0, b, b ^ jnp.int32(_IMAX_I))


def _score_call(qf, mem_keys):
    return pl.pallas_call(
        _score_body,
        grid=(_NB,),
        in_specs=[
            pl.BlockSpec((_NQ, _DH), lambda i: (0, 0)),
            pl.BlockSpec((_W, _DH), lambda i: (i, 0)),
        ],
        out_specs=pl.BlockSpec((_NQ, _W), lambda i: (0, i)),
        out_shape=jax.ShapeDtypeStruct((_NQ, _NPAD), jnp.int32),
    )(qf, mem_keys)


def _topk_body(keys_hbm, mk_hbm, mv_hbm, ksel_hbm, vsel_hbm,
               sv, cs, ci, tiv, kr, vr, ts, ti, sem):
    w = lax.axis_index("s") * 2 + lax.axis_index("c")
    pltpu.sync_copy(keys_hbm.at[w], sv)

    def initb(j, c):
        ts[j] = jnp.int32(_IMIN_I)
        ti[j] = jnp.int32(_IMAX_I)
        return c

    lax.fori_loop(0, _K, initb, jnp.int32(0))

    def _better(s1, i1, s2, i2):
        return (s1 > s2) | ((s1 == s2) & (i1 < i2))

    def _insert(s, ii, enabled):
        def do():
            def cond_fn(p):
                pm = jnp.maximum(p - 1, 0)
                return (p > 0) & _better(s, ii, ts[pm], ti[pm])

            def step(p):
                ts[p] = ts[p - 1]
                ti[p] = ti[p - 1]
                return p - 1

            p = lax.while_loop(cond_fn, step, jnp.int32(_K - 1))
            ts[p] = s
            ti[p] = ii

        pl.when(enabled & _better(s, ii, ts[_K - 1], ti[_K - 1]))(do)

    def _compact(wp):
        for vi in range(4):
            cv = cs[pl.ds(vi * 16, 16)]
            civ = ci[pl.ds(vi * 16, 16)]
            for j in range(16):
                g = vi * 16 + j
                _insert(cv[j], civ[j], jnp.int32(g) < wp)
        return ts[_K - 1]

    def scan_body(i, carry):
        thr, wp = carry
        v = sv[pl.ds(i * 16, 16)]
        m = v >= thr
        cnt = jnp.sum(jnp.where(m, jnp.int32(1), jnp.int32(0)))

        def app(args):
            thr, wp = args
            plsc.store_compressed(cs.at[pl.ds(wp, 16)], v, mask=m)
            iv = lax.iota(jnp.int32, 16) + i * 16
            plsc.store_compressed(ci.at[pl.ds(wp, 16)], iv, mask=m)
            wp2 = wp + cnt

            def comp(_):
                return _compact(wp2), jnp.int32(0)

            return lax.cond(wp2 > _CAP, comp, lambda a: a, (thr, wp2))

        return lax.cond(cnt > 0, app, lambda a: a, (thr, wp))

    thr, wp = lax.fori_loop(0, _N // 16, scan_body,
                            (jnp.int32(_IMIN_I), jnp.int32(0)))

    def final():
        _compact(wp)

    pl.when(wp > 0)(final)

    lanes = lax.iota(jnp.int32, 16)
    for half in range(2):
        v = jnp.zeros((16,), jnp.int32)
        for l in range(16):
            v = jnp.where(lanes == l, ti[half * 16 + l], v)
        tiv[pl.ds(half * 16, 16)] = v

    pltpu.async_copy(mk_hbm.at[tiv], kr, sem).wait()
    pltpu.async_copy(mv_hbm.at[tiv], vr, sem).wait()
    pltpu.sync_copy(kr, ksel_hbm.at[w])
    pltpu.sync_copy(vr, vsel_hbm.at[w])


def _topk_call(keys, mem_keys, mem_values):
    mesh = plsc.VectorSubcoreMesh(core_axis_name="c", subcore_axis_name="s")
    fn = pl.kernel(
        _topk_body,
        mesh=mesh,
        compiler_params=pltpu.CompilerParams(needs_layout_passes=False),
        out_type=[
            jax.ShapeDtypeStruct((_NQ, _K, _DH), jnp.float32),
            jax.ShapeDtypeStruct((_NQ, _K, _DH), jnp.float32),
        ],
        scratch_types=[
            pltpu.VMEM((_NPAD,), jnp.int32),
            pltpu.VMEM((64,), jnp.int32),
            pltpu.VMEM((64,), jnp.int32),
            pltpu.VMEM((_K,), jnp.int32),
            pltpu.VMEM((_K, _DH), jnp.float32),
            pltpu.VMEM((_K, _DH), jnp.float32),
            pltpu.SMEM((_K,), jnp.int32),
            pltpu.SMEM((_K,), jnp.int32),
            pltpu.SemaphoreType.DMA,
        ],
    )
    return fn(keys, mem_keys, mem_values)


def _attn_body(q_ref, kc_ref, vc_ref, ks_ref, vs_ref, o_ref, w_ref):
    qb = q_ref[0, 0]
    kc = kc_ref[0, 0]
    vc = vc_ref[0, 0]
    ks = ks_ref[0]
    vs = vs_ref[0]
    scale = 1.0 / math.sqrt(_DH)
    s_ctx = lax.dot_general(qb, kc, (((1,), (1,)), ((), ())),
                            preferred_element_type=jnp.float32) * scale
    qb16 = qb.astype(jnp.bfloat16).astype(jnp.float32)
    ks16 = ks.astype(jnp.bfloat16).astype(jnp.float32)
    s_mem = jnp.sum(qb16[:, None, :] * ks16, axis=-1) * scale
    m = jnp.maximum(jnp.max(s_ctx, axis=-1), jnp.max(s_mem, axis=-1))
    e_ctx = jnp.exp(s_ctx - m[:, None])
    e_mem = jnp.exp(s_mem - m[:, None])
    rd = 1.0 / (jnp.sum(e_ctx, axis=-1) + jnp.sum(e_mem, axis=-1))
    w_ctx = e_ctx * rd[:, None]
    w_mem = e_mem * rd[:, None]
    w_ref[0, 0, :, :_TK] = w_ctx
    w_ref[0, 0, :, _TK:] = w_mem
    wm16 = w_mem.astype(jnp.bfloat16).astype(jnp.float32)
    vs16 = vs.astype(jnp.bfloat16).astype(jnp.float32)
    o_ref[0, 0] = (lax.dot_general(w_ctx, vc, (((1,), (0,)), ((), ())),
                                   preferred_element_type=jnp.float32)
                   + jnp.sum(wm16[:, :, None] * vs16, axis=1))


def _attn_call(q, k_ctx, v_ctx, ksel, vsel):
    return pl.pallas_call(
        _attn_body,
        grid=(_B, _H),
        in_specs=[
            pl.BlockSpec((1, 1, _TQ, _DH), lambda b, h: (b, h, 0, 0)),
            pl.BlockSpec((1, 1, _TK, _DH), lambda b, h: (b, h, 0, 0)),
            pl.BlockSpec((1, 1, _TK, _DH), lambda b, h: (b, h, 0, 0)),
            pl.BlockSpec((1, _TQ, _K, _DH), lambda b, h: (b, 0, 0, 0)),
            pl.BlockSpec((1, _TQ, _K, _DH), lambda b, h: (b, 0, 0, 0)),
        ],
        out_specs=[
            pl.BlockSpec((1, 1, _TQ, _DH), lambda b, h: (b, h, 0, 0)),
            pl.BlockSpec((1, 1, _TQ, _TK + _K), lambda b, h: (b, h, 0, 0)),
        ],
        out_shape=[
            jax.ShapeDtypeStruct((_B, _H, _TQ, _DH), jnp.float32),
            jax.ShapeDtypeStruct((_B, _H, _TQ, _TK + _K), jnp.float32),
        ],
    )(q, k_ctx, v_ctx, ksel, vsel)


def kernel(q, k_ctx, v_ctx, q_for_mem, mem_keys, mem_values):
    qf = q_for_mem.reshape(_NQ, _DH)
    keys = _score_call(qf, mem_keys)
    ksel, vsel = _topk_call(keys, mem_keys, mem_values)
    ksel = ksel.reshape(_B, _TQ, _K, _DH)
    vsel = vsel.reshape(_B, _TQ, _K, _DH)
    return _attn_call(q, k_ctx, v_ctx, ksel, vsel)


# speedup vs baseline: 9.0017x; 1.3810x over previous
import math

import jax
import jax.numpy as jnp
from jax import lax
from jax.experimental import pallas as pl
from jax.experimental.pallas import tpu as pltpu
from jax.experimental.pallas import tpu_sc as plsc

_B, _H, _TQ, _DH = 8, 16, 4, 128
_TK = 2048
_N = 100000
_K = 32
_NQ = _B * _TQ

_W = 6400
_NB = 16
_NPAD = _W * _NB

_CAP = 48
_IMIN_I = -(2 ** 31)
_IMAX_I = 2 ** 31 - 1


def _score_body(qf_ref, mk_ref, s_ref):
    s = lax.dot_general(
        qf_ref[...], mk_ref[...], (((1,), (1,)), ((), ())),
        preferred_element_type=jnp.float32)
    b = jax.lax.bitcast_convert_type(s, jnp.int32)
    s_ref[...] = jnp.where(b >= 0, b, b ^ jnp.int32(_IMAX_I))


def _score_call(qf, mem_keys):
    return pl.pallas_call(
        _score_body,
        grid=(_NB,),
        in_specs=[
            pl.BlockSpec((_NQ, _DH), lambda i: (0, 0)),
            pl.BlockSpec((_W, _DH), lambda i: (i, 0)),
        ],
        out_specs=pl.BlockSpec((_NQ, _W), lambda i: (0, i)),
        out_shape=jax.ShapeDtypeStruct((_NQ, _NPAD), jnp.int32),
    )(qf, mem_keys)


def _topk_body(keys_hbm, mk_hbm, mv_hbm, ksel_hbm, vsel_hbm,
               sv, cs, ci, tiv, kr, vr, ts, ti, sem):
    w = lax.axis_index("s") * 2 + lax.axis_index("c")
    pltpu.sync_copy(keys_hbm.at[w], sv)

    def initb(j, c):
        ts[j] = jnp.int32(_IMIN_I)
        ti[j] = jnp.int32(_IMAX_I)
        return c

    lax.fori_loop(0, _K, initb, jnp.int32(0))

    def _better(s1, i1, s2, i2):
        return (s1 > s2) | ((s1 == s2) & (i1 < i2))

    def _insert(s, ii):
        def do():
            def cond_fn(p):
                pm = jnp.maximum(p - 1, 0)
                return (p > 0) & _better(s, ii, ts[pm], ti[pm])

            def step(p):
                ts[p] = ts[p - 1]
                ti[p] = ti[p - 1]
                return p - 1

            p = lax.while_loop(cond_fn, step, jnp.int32(_K - 1))
            ts[p] = s
            ti[p] = ii

        pl.when(_better(s, ii, ts[_K - 1], ti[_K - 1]))(do)

    def _compact(wp):
        def ins(j, c):
            jv = jnp.full((16,), j, jnp.int32)
            _insert(plsc.load_gather(cs, [jv])[0],
                    plsc.load_gather(ci, [jv])[0])
            return c

        lax.fori_loop(0, wp, ins, jnp.int32(0))
        return ts[_K - 1]

    lanes16 = lax.iota(jnp.int32, 16)

    def _append_one(v, m, iv, carry):
        thr, wp = carry
        cnt = plsc.all_reduce_population_count(m)[0]

        def app(args):
            thr, wp = args
            plsc.store_compressed(cs.at[pl.ds(wp, 16)], v, mask=m)
            plsc.store_compressed(ci.at[pl.ds(wp, 16)], iv, mask=m)
            wp2 = wp + cnt

            def comp(_):
                nthr = _compact(wp2)
                return jnp.full((16,), nthr, jnp.int32), jnp.int32(0)

            return lax.cond(wp2 > _CAP, comp, lambda a: a, (thr, wp2))

        return lax.cond(cnt > 0, app, lambda a: a, (thr, wp))

    _UNROLL = 5

    def scan_body(i, carry):
        thr, wp = carry
        base = i * (16 * _UNROLL)
        vs_ = [sv[pl.ds(base + 16 * u, 16)] for u in range(_UNROLL)]
        ms_ = [v >= thr for v in vs_]
        any_m = ms_[0]
        for u in range(1, _UNROLL):
            any_m = any_m | ms_[u]
        cnt_any = plsc.all_reduce_population_count(any_m)[0]

        def app(args):
            c = args
            for u in range(_UNROLL):
                c = _append_one(vs_[u], ms_[u], lanes16 + (base + 16 * u), c)
            return c

        return lax.cond(cnt_any > 0, app, lambda a: a, (thr, wp))

    thr, wp = lax.fori_loop(0, _N // (16 * _UNROLL), scan_body,
                            (jnp.full((16,), jnp.int32(_IMIN_I), jnp.int32),
                             jnp.int32(0)))

    def final():
        _compact(wp)

    pl.when(wp > 0)(final)

    lanes = lax.iota(jnp.int32, 16)
    for half in range(2):
        v = jnp.zeros((16,), jnp.int32)
        for l in range(16):
            v = jnp.where(lanes == l, ti[half * 16 + l], v)
        tiv[pl.ds(half * 16, 16)] = v

    pltpu.async_copy(mk_hbm.at[tiv], kr, sem).wait()
    pltpu.async_copy(mv_hbm.at[tiv], vr, sem).wait()
    pltpu.sync_copy(kr, ksel_hbm.at[w])
    pltpu.sync_copy(vr, vsel_hbm.at[w])


def _topk_call(keys, mem_keys, mem_values):
    mesh = plsc.VectorSubcoreMesh(core_axis_name="c", subcore_axis_name="s")
    fn = pl.kernel(
        _topk_body,
        mesh=mesh,
        compiler_params=pltpu.CompilerParams(needs_layout_passes=False),
        out_type=[
            jax.ShapeDtypeStruct((_NQ, _K, _DH), jnp.float32),
            jax.ShapeDtypeStruct((_NQ, _K, _DH), jnp.float32),
        ],
        scratch_types=[
            pltpu.VMEM((_NPAD,), jnp.int32),
            pltpu.VMEM((64,), jnp.int32),
            pltpu.VMEM((64,), jnp.int32),
            pltpu.VMEM((_K,), jnp.int32),
            pltpu.VMEM((_K, _DH), jnp.float32),
            pltpu.VMEM((_K, _DH), jnp.float32),
            pltpu.SMEM((_K,), jnp.int32),
            pltpu.SMEM((_K,), jnp.int32),
            pltpu.SemaphoreType.DMA,
        ],
    )
    return fn(keys, mem_keys, mem_values)


def _attn_body(q_ref, kc_ref, vc_ref, ks_ref, vs_ref, o_ref, w_ref):
    qb = q_ref[0, 0]
    kc = kc_ref[0, 0]
    vc = vc_ref[0, 0]
    ks = ks_ref[0]
    vs = vs_ref[0]
    scale = 1.0 / math.sqrt(_DH)
    s_ctx = lax.dot_general(qb, kc, (((1,), (1,)), ((), ())),
                            preferred_element_type=jnp.float32) * scale
    qb16 = qb.astype(jnp.bfloat16).astype(jnp.float32)
    ks16 = ks.astype(jnp.bfloat16).astype(jnp.float32)
    s_mem = jnp.sum(qb16[:, None, :] * ks16, axis=-1) * scale
    m = jnp.maximum(jnp.max(s_ctx, axis=-1), jnp.max(s_mem, axis=-1))
    e_ctx = jnp.exp(s_ctx - m[:, None])
    e_mem = jnp.exp(s_mem - m[:, None])
    rd = 1.0 / (jnp.sum(e_ctx, axis=-1) + jnp.sum(e_mem, axis=-1))
    w_ctx = e_ctx * rd[:, None]
    w_mem = e_mem * rd[:, None]
    w_ref[0, 0, :, :_TK] = w_ctx
    w_ref[0, 0, :, _TK:] = w_mem
    wm16 = w_mem.astype(jnp.bfloat16).astype(jnp.float32)
    vs16 = vs.astype(jnp.bfloat16).astype(jnp.float32)
    o_ref[0, 0] = (lax.dot_general(w_ctx, vc, (((1,), (0,)), ((), ())),
                                   preferred_element_type=jnp.float32)
                   + jnp.sum(wm16[:, :, None] * vs16, axis=1))


def _attn_call(q, k_ctx, v_ctx, ksel, vsel):
    return pl.pallas_call(
        _attn_body,
        grid=(_B, _H),
        in_specs=[
            pl.BlockSpec((1, 1, _TQ, _DH), lambda b, h: (b, h, 0, 0)),
            pl.BlockSpec((1, 1, _TK, _DH), lambda b, h: (b, h, 0, 0)),
            pl.BlockSpec((1, 1, _TK, _DH), lambda b, h: (b, h, 0, 0)),
            pl.BlockSpec((1, _TQ, _K, _DH), lambda b, h: (b, 0, 0, 0)),
            pl.BlockSpec((1, _TQ, _K, _DH), lambda b, h: (b, 0, 0, 0)),
        ],
        out_specs=[
            pl.BlockSpec((1, 1, _TQ, _DH), lambda b, h: (b, h, 0, 0)),
            pl.BlockSpec((1, 1, _TQ, _TK + _K), lambda b, h: (b, h, 0, 0)),
        ],
        out_shape=[
            jax.ShapeDtypeStruct((_B, _H, _TQ, _DH), jnp.float32),
            jax.ShapeDtypeStruct((_B, _H, _TQ, _TK + _K), jnp.float32),
        ],
    )(q, k_ctx, v_ctx, ksel, vsel)


def kernel(q, k_ctx, v_ctx, q_for_mem, mem_keys, mem_values):
    qf = q_for_mem.reshape(_NQ, _DH)
    keys = _score_call(qf, mem_keys)
    ksel, vsel = _topk_call(keys, mem_keys, mem_values)
    ksel = ksel.reshape(_B, _TQ, _K, _DH)
    vsel = vsel.reshape(_B, _TQ, _K, _DH)
    return _attn_call(q, k_ctx, v_ctx, ksel, vsel)


# speedup vs baseline: 9.3345x; 1.0370x over previous
import math

import jax
import jax.numpy as jnp
from jax import lax
from jax.experimental import pallas as pl
from jax.experimental.pallas import tpu as pltpu
from jax.experimental.pallas import tpu_sc as plsc

_B, _H, _TQ, _DH = 8, 16, 4, 128
_TK = 2048
_N = 100000
_K = 32
_NQ = _B * _TQ

_W = 16384
_NB = 7
_NPAD = _W * _NB

_CAP = 48
_IMIN_I = -(2 ** 31)
_IMAX_I = 2 ** 31 - 1


_BW = 128
_NBM = _NPAD // _BW


def _score_body(qf_ref, mk_ref, s_ref, bm_ref):
    s = lax.dot_general(
        qf_ref[...], mk_ref[...], (((1,), (1,)), ((), ())),
        preferred_element_type=jnp.float32)
    b = jax.lax.bitcast_convert_type(s, jnp.int32)
    key = jnp.where(b >= 0, b, b ^ jnp.int32(_IMAX_I))
    j = pl.program_id(0)
    gcol = j * _W + lax.broadcasted_iota(jnp.int32, (_NQ, _W), 1)
    key = jnp.where(gcol < _N, key, jnp.int32(_IMIN_I))
    s_ref[...] = key
    for k in range(_W // _BW):
        bm_ref[:, k:k + 1] = jnp.max(key[:, k * _BW:(k + 1) * _BW],
                                     axis=-1, keepdims=True)


def _score_call(qf, mem_keys):
    return pl.pallas_call(
        _score_body,
        grid=(_NB,),
        in_specs=[
            pl.BlockSpec((_NQ, _DH), lambda i: (0, 0)),
            pl.BlockSpec((_W, _DH), lambda i: (i, 0)),
        ],
        out_specs=[
            pl.BlockSpec((_NQ, _W), lambda i: (0, i)),
            pl.BlockSpec((_NQ, 128), lambda i: (0, i)),
        ],
        out_shape=[
            jax.ShapeDtypeStruct((_NQ, _NPAD), jnp.int32),
            jax.ShapeDtypeStruct((_NQ, _NBM), jnp.int32),
        ],
    )(qf, mem_keys)


def _topk_body(keys_hbm, bmax_hbm, mk_hbm, mv_hbm, ksel_hbm, vsel_hbm,
               sv, bv, cs, ci, tiv, kr, vr, ts, ti, sem):
    w = lax.axis_index("s") * 2 + lax.axis_index("c")
    pltpu.sync_copy(keys_hbm.at[w], sv)
    pltpu.sync_copy(bmax_hbm.at[w], bv)

    def initb(j, c):
        ts[j] = jnp.int32(_IMIN_I)
        ti[j] = jnp.int32(_IMAX_I)
        return c

    lax.fori_loop(0, _K, initb, jnp.int32(0))

    def _better(s1, i1, s2, i2):
        return (s1 > s2) | ((s1 == s2) & (i1 < i2))

    def _insert(s, ii):
        def do():
            def cond_fn(p):
                pm = jnp.maximum(p - 1, 0)
                return (p > 0) & _better(s, ii, ts[pm], ti[pm])

            def step(p):
                ts[p] = ts[p - 1]
                ti[p] = ti[p - 1]
                return p - 1

            p = lax.while_loop(cond_fn, step, jnp.int32(_K - 1))
            ts[p] = s
            ti[p] = ii

        pl.when(_better(s, ii, ts[_K - 1], ti[_K - 1]))(do)

    def _compact(wp):
        def ins(j, c):
            jv = jnp.full((16,), j, jnp.int32)
            _insert(plsc.load_gather(cs, [jv])[0],
                    plsc.load_gather(ci, [jv])[0])
            return c

        lax.fori_loop(0, wp, ins, jnp.int32(0))
        return ts[_K - 1]

    lanes16 = lax.iota(jnp.int32, 16)

    def _append_one(v, m, iv, carry):
        thr, wp = carry
        cnt = plsc.all_reduce_population_count(m)[0]

        def app(args):
            thr, wp = args
            plsc.store_compressed(cs.at[pl.ds(wp, 16)], v, mask=m)
            plsc.store_compressed(ci.at[pl.ds(wp, 16)], iv, mask=m)
            wp2 = wp + cnt

            def comp(_):
                nthr = _compact(wp2)
                return jnp.full((16,), nthr, jnp.int32), jnp.int32(0)

            return lax.cond(wp2 > _CAP, comp, lambda a: a, (thr, wp2))

        return lax.cond(cnt > 0, app, lambda a: a, (thr, wp))

    def _visit_block(blk, carry):
        base = blk * _BW
        c = carry
        for u in range(_BW // 16):
            thr, wp = c
            v = sv[pl.ds(base + 16 * u, 16)]
            m = v >= thr
            c = _append_one(v, m, lanes16 + (base + 16 * u), c)
        return c

    def screen_body(si, carry):
        thr, wp = carry
        bm = bv[pl.ds(si * 16, 16)]
        m = bm >= thr

        def visit(args):
            def w_cond(st):
                mm, thr, wp = st
                return plsc.all_reduce_population_count(mm)[0] > 0

            def w_body(st):
                mm, thr, wp = st
                p = plsc.all_reduce_ffs(mm)[0]
                thr, wp = _visit_block(si * 16 + p, (thr, wp))
                return mm & (lanes16 != p) & (bm >= thr), thr, wp

            _, thr, wp = lax.while_loop(w_cond, w_body, (args[0], args[1], args[2]))
            return thr, wp

        cnt = plsc.all_reduce_population_count(m)[0]
        return lax.cond(cnt > 0, lambda a: visit((m, a[0], a[1])),
                        lambda a: a, (thr, wp))

    thr, wp = lax.fori_loop(0, _NBM // 16, screen_body,
                            (jnp.full((16,), jnp.int32(_IMIN_I), jnp.int32),
                             jnp.int32(0)))

    def final():
        _compact(wp)

    pl.when(wp > 0)(final)

    lanes = lax.iota(jnp.int32, 16)
    for half in range(2):
        v = jnp.zeros((16,), jnp.int32)
        for l in range(16):
            v = jnp.where(lanes == l, ti[half * 16 + l], v)
        tiv[pl.ds(half * 16, 16)] = v

    pltpu.async_copy(mk_hbm.at[tiv], kr, sem).wait()
    pltpu.async_copy(mv_hbm.at[tiv], vr, sem).wait()
    pltpu.sync_copy(kr, ksel_hbm.at[w])
    pltpu.sync_copy(vr, vsel_hbm.at[w])


def _topk_call(keys, bmax, mem_keys, mem_values):
    mesh = plsc.VectorSubcoreMesh(core_axis_name="c", subcore_axis_name="s")
    fn = pl.kernel(
        _topk_body,
        mesh=mesh,
        compiler_params=pltpu.CompilerParams(needs_layout_passes=False),
        out_type=[
            jax.ShapeDtypeStruct((_NQ, _K, _DH), jnp.float32),
            jax.ShapeDtypeStruct((_NQ, _K, _DH), jnp.float32),
        ],
        scratch_types=[
            pltpu.VMEM((_NPAD,), jnp.int32),
            pltpu.VMEM((_NBM,), jnp.int32),
            pltpu.VMEM((64,), jnp.int32),
            pltpu.VMEM((64,), jnp.int32),
            pltpu.VMEM((_K,), jnp.int32),
            pltpu.VMEM((_K, _DH), jnp.float32),
            pltpu.VMEM((_K, _DH), jnp.float32),
            pltpu.SMEM((_K,), jnp.int32),
            pltpu.SMEM((_K,), jnp.int32),
            pltpu.SemaphoreType.DMA,
        ],
    )
    return fn(keys, bmax, mem_keys, mem_values)


def _attn_body(q_ref, kc_ref, vc_ref, ks_ref, vs_ref, o_ref, w_ref):
    qb = q_ref[0, 0]
    kc = kc_ref[0, 0]
    vc = vc_ref[0, 0]
    ks = ks_ref[0]
    vs = vs_ref[0]
    scale = 1.0 / math.sqrt(_DH)
    s_ctx = lax.dot_general(qb, kc, (((1,), (1,)), ((), ())),
                            preferred_element_type=jnp.float32) * scale
    qb16 = qb.astype(jnp.bfloat16).astype(jnp.float32)
    ks16 = ks.astype(jnp.bfloat16).astype(jnp.float32)
    s_mem = jnp.sum(qb16[:, None, :] * ks16, axis=-1) * scale
    m = jnp.maximum(jnp.max(s_ctx, axis=-1), jnp.max(s_mem, axis=-1))
    e_ctx = jnp.exp(s_ctx - m[:, None])
    e_mem = jnp.exp(s_mem - m[:, None])
    rd = 1.0 / (jnp.sum(e_ctx, axis=-1) + jnp.sum(e_mem, axis=-1))
    w_ctx = e_ctx * rd[:, None]
    w_mem = e_mem * rd[:, None]
    w_ref[0, 0, :, :_TK] = w_ctx
    w_ref[0, 0, :, _TK:] = w_mem
    wm16 = w_mem.astype(jnp.bfloat16).astype(jnp.float32)
    vs16 = vs.astype(jnp.bfloat16).astype(jnp.float32)
    o_ref[0, 0] = (lax.dot_general(w_ctx, vc, (((1,), (0,)), ((), ())),
                                   preferred_element_type=jnp.float32)
                   + jnp.sum(wm16[:, :, None] * vs16, axis=1))


def _attn_call(q, k_ctx, v_ctx, ksel, vsel):
    return pl.pallas_call(
        _attn_body,
        grid=(_B, _H),
        in_specs=[
            pl.BlockSpec((1, 1, _TQ, _DH), lambda b, h: (b, h, 0, 0)),
            pl.BlockSpec((1, 1, _TK, _DH), lambda b, h: (b, h, 0, 0)),
            pl.BlockSpec((1, 1, _TK, _DH), lambda b, h: (b, h, 0, 0)),
            pl.BlockSpec((1, _TQ, _K, _DH), lambda b, h: (b, 0, 0, 0)),
            pl.BlockSpec((1, _TQ, _K, _DH), lambda b, h: (b, 0, 0, 0)),
        ],
        out_specs=[
            pl.BlockSpec((1, 1, _TQ, _DH), lambda b, h: (b, h, 0, 0)),
            pl.BlockSpec((1, 1, _TQ, _TK + _K), lambda b, h: (b, h, 0, 0)),
        ],
        out_shape=[
            jax.ShapeDtypeStruct((_B, _H, _TQ, _DH), jnp.float32),
            jax.ShapeDtypeStruct((_B, _H, _TQ, _TK + _K), jnp.float32),
        ],
    )(q, k_ctx, v_ctx, ksel, vsel)


def kernel(q, k_ctx, v_ctx, q_for_mem, mem_keys, mem_values):
    qf = q_for_mem.reshape(_NQ, _DH)
    keys, bmax = _score_call(qf, mem_keys)
    ksel, vsel = _topk_call(keys, bmax, mem_keys, mem_values)
    ksel = ksel.reshape(_B, _TQ, _K, _DH)
    vsel = vsel.reshape(_B, _TQ, _K, _DH)
    return _attn_call(q, k_ctx, v_ctx, ksel, vsel)


# speedup vs baseline: 15.5847x; 1.6696x over previous
import math

import jax
import jax.numpy as jnp
from jax import lax
from jax.experimental import pallas as pl
from jax.experimental.pallas import tpu as pltpu
from jax.experimental.pallas import tpu_sc as plsc

_B, _H, _TQ, _DH = 8, 16, 4, 128
_TK = 2048
_N = 100000
_K = 32
_NQ = _B * _TQ

_W = 16384
_NB = 7
_NPAD = _W * _NB

_CAP = 48
_IMIN_I = -(2 ** 31)
_IMAX_I = 2 ** 31 - 1


_BW = 128
_NBM = _NPAD // _BW


def _score_body(qf_ref, mk_ref, s_ref, bm_ref):
    s = lax.dot_general(
        qf_ref[...], mk_ref[...], (((1,), (1,)), ((), ())),
        preferred_element_type=jnp.float32)
    b = jax.lax.bitcast_convert_type(s, jnp.int32)
    key = jnp.where(b >= 0, b, b ^ jnp.int32(_IMAX_I))
    j = pl.program_id(0)
    gcol = j * _W + lax.broadcasted_iota(jnp.int32, (_NQ, _W), 1)
    key = jnp.where(gcol < _N, key, jnp.int32(_IMIN_I))
    s_ref[...] = key
    for k in range(_W // _BW):
        bm_ref[:, k:k + 1] = jnp.max(key[:, k * _BW:(k + 1) * _BW],
                                     axis=-1, keepdims=True)


def _score_call(qf, mem_keys):
    return pl.pallas_call(
        _score_body,
        grid=(_NB,),
        in_specs=[
            pl.BlockSpec((_NQ, _DH), lambda i: (0, 0)),
            pl.BlockSpec((_W, _DH), lambda i: (i, 0)),
        ],
        out_specs=[
            pl.BlockSpec((_NQ, _W), lambda i: (0, i)),
            pl.BlockSpec((_NQ, 128), lambda i: (0, i)),
        ],
        out_shape=[
            jax.ShapeDtypeStruct((_NQ, _NPAD), jnp.int32),
            jax.ShapeDtypeStruct((_NQ, _NBM), jnp.int32),
        ],
    )(qf, mem_keys)


def _topk_body(keys_hbm, bmax_hbm, mk_hbm, mv_hbm, ksel_hbm, vsel_hbm,
               sv, bv, cs, ci, tiv, kr, vr, ts, ti, sem):
    w = lax.axis_index("s") * 2 + lax.axis_index("c")
    pltpu.sync_copy(keys_hbm.at[w], sv)
    pltpu.sync_copy(bmax_hbm.at[w], bv)

    def initb(j, c):
        ts[j] = jnp.int32(_IMIN_I)
        ti[j] = jnp.int32(_IMAX_I)
        return c

    lax.fori_loop(0, _K, initb, jnp.int32(0))

    def _better(s1, i1, s2, i2):
        return (s1 > s2) | ((s1 == s2) & (i1 < i2))

    def _insert(s, ii):
        def do():
            def cond_fn(p):
                pm = jnp.maximum(p - 1, 0)
                return (p > 0) & _better(s, ii, ts[pm], ti[pm])

            def step(p):
                ts[p] = ts[p - 1]
                ti[p] = ti[p - 1]
                return p - 1

            p = lax.while_loop(cond_fn, step, jnp.int32(_K - 1))
            ts[p] = s
            ti[p] = ii

        pl.when(_better(s, ii, ts[_K - 1], ti[_K - 1]))(do)

    def _compact(wp):
        def ins(j, c):
            jv = jnp.full((16,), j, jnp.int32)
            _insert(plsc.load_gather(cs, [jv])[0],
                    plsc.load_gather(ci, [jv])[0])
            return c

        lax.fori_loop(0, wp, ins, jnp.int32(0))
        return ts[_K - 1]

    lanes16 = lax.iota(jnp.int32, 16)

    def _append_one(v, m, iv, carry):
        thr, wp = carry
        cnt = plsc.all_reduce_population_count(m)[0]

        def app(args):
            thr, wp = args
            plsc.store_compressed(cs.at[pl.ds(wp, 16)], v, mask=m)
            plsc.store_compressed(ci.at[pl.ds(wp, 16)], iv, mask=m)
            wp2 = wp + cnt

            def comp(_):
                nthr = _compact(wp2)
                return jnp.full((16,), nthr, jnp.int32), jnp.int32(0)

            return lax.cond(wp2 > _CAP, comp, lambda a: a, (thr, wp2))

        return lax.cond(cnt > 0, app, lambda a: a, (thr, wp))

    def _visit_block(blk, carry):
        base = blk * _BW
        c = carry
        for u in range(_BW // 16):
            thr, wp = c
            v = sv[pl.ds(base + 16 * u, 16)]
            m = v >= thr
            c = _append_one(v, m, lanes16 + (base + 16 * u), c)
        return c

    def screen_body(si, carry):
        thr, wp = carry
        bm = bv[pl.ds(si * 16, 16)]
        m = bm >= thr

        def visit(args):
            def w_cond(st):
                mm, thr, wp = st
                return plsc.all_reduce_population_count(mm)[0] > 0

            def w_body(st):
                mm, thr, wp = st
                p = plsc.all_reduce_ffs(mm)[0]
                thr, wp = _visit_block(si * 16 + p, (thr, wp))
                return mm & (lanes16 != p) & (bm >= thr), thr, wp

            _, thr, wp = lax.while_loop(w_cond, w_body, (args[0], args[1], args[2]))
            return thr, wp

        cnt = plsc.all_reduce_population_count(m)[0]
        return lax.cond(cnt > 0, lambda a: visit((m, a[0], a[1])),
                        lambda a: a, (thr, wp))

    thr, wp = lax.fori_loop(0, _NBM // 16, screen_body,
                            (jnp.full((16,), jnp.int32(_IMIN_I), jnp.int32),
                             jnp.int32(0)))

    def final():
        _compact(wp)

    pl.when(wp > 0)(final)

    lanes = lax.iota(jnp.int32, 16)
    for half in range(2):
        v = jnp.zeros((16,), jnp.int32)
        for l in range(16):
            v = jnp.where(lanes == l, ti[half * 16 + l], v)
        tiv[pl.ds(half * 16, 16)] = v

    pltpu.async_copy(mk_hbm.at[tiv], kr, sem).wait()
    pltpu.async_copy(mv_hbm.at[tiv], vr, sem).wait()
    pltpu.sync_copy(kr, ksel_hbm.at[w])
    pltpu.sync_copy(vr, vsel_hbm.at[w])


def _topk_call(keys, bmax, mem_keys, mem_values):
    mesh = plsc.VectorSubcoreMesh(core_axis_name="c", subcore_axis_name="s")
    fn = pl.kernel(
        _topk_body,
        mesh=mesh,
        compiler_params=pltpu.CompilerParams(needs_layout_passes=False),
        out_type=[
            jax.ShapeDtypeStruct((_NQ, _K, _DH), jnp.float32),
            jax.ShapeDtypeStruct((_NQ, _K, _DH), jnp.float32),
        ],
        scratch_types=[
            pltpu.VMEM((_NPAD,), jnp.int32),
            pltpu.VMEM((_NBM,), jnp.int32),
            pltpu.VMEM((64,), jnp.int32),
            pltpu.VMEM((64,), jnp.int32),
            pltpu.VMEM((_K,), jnp.int32),
            pltpu.VMEM((_K, _DH), jnp.float32),
            pltpu.VMEM((_K, _DH), jnp.float32),
            pltpu.SMEM((_K,), jnp.int32),
            pltpu.SMEM((_K,), jnp.int32),
            pltpu.SemaphoreType.DMA,
        ],
    )
    return fn(keys, bmax, mem_keys, mem_values)


def _ctx_body(q_ref, kc_ref, vc_ref, ec_ref, oc_ref, mx_ref, lx_ref):
    qb = q_ref[0, 0]
    kc = kc_ref[0, 0]
    vc = vc_ref[0, 0]
    scale = 1.0 / math.sqrt(_DH)
    s_ctx = lax.dot_general(qb, kc, (((1,), (1,)), ((), ())),
                            preferred_element_type=jnp.float32) * scale
    mc = jnp.max(s_ctx, axis=-1, keepdims=True)
    e = jnp.exp(s_ctx - mc)
    ec_ref[0, 0] = e
    oc_ref[0, 0] = lax.dot_general(e, vc, (((1,), (0,)), ((), ())),
                                   preferred_element_type=jnp.float32)
    mx_ref[0, 0] = mc
    lx_ref[0, 0] = jnp.sum(e, axis=-1, keepdims=True)


def _ctx_call(q, k_ctx, v_ctx):
    return pl.pallas_call(
        _ctx_body,
        grid=(_B, _H),
        in_specs=[
            pl.BlockSpec((1, 1, _TQ, _DH), lambda b, h: (b, h, 0, 0)),
            pl.BlockSpec((1, 1, _TK, _DH), lambda b, h: (b, h, 0, 0)),
            pl.BlockSpec((1, 1, _TK, _DH), lambda b, h: (b, h, 0, 0)),
        ],
        out_specs=[
            pl.BlockSpec((1, 1, _TQ, _TK), lambda b, h: (b, h, 0, 0)),
            pl.BlockSpec((1, 1, _TQ, _DH), lambda b, h: (b, h, 0, 0)),
            pl.BlockSpec((1, 1, _TQ, 1), lambda b, h: (b, h, 0, 0)),
            pl.BlockSpec((1, 1, _TQ, 1), lambda b, h: (b, h, 0, 0)),
        ],
        out_shape=[
            jax.ShapeDtypeStruct((_B, _H, _TQ, _TK), jnp.float32),
            jax.ShapeDtypeStruct((_B, _H, _TQ, _DH), jnp.float32),
            jax.ShapeDtypeStruct((_B, _H, _TQ, 1), jnp.float32),
            jax.ShapeDtypeStruct((_B, _H, _TQ, 1), jnp.float32),
        ],
    )(q, k_ctx, v_ctx)


def _merge_body(q_ref, ks_ref, vs_ref, ec_ref, oc_ref, mx_ref, lx_ref,
                o_ref, w_ref):
    qb = q_ref[0]
    ks = ks_ref[0]
    vs = vs_ref[0]
    ec = ec_ref[0]
    oc = oc_ref[0]
    mx = mx_ref[0]
    lx = lx_ref[0]
    scale = 1.0 / math.sqrt(_DH)
    qb16 = qb.astype(jnp.bfloat16).astype(jnp.float32)
    ks16 = ks.astype(jnp.bfloat16).astype(jnp.float32)
    s_mem = jnp.sum(qb16[:, :, None, :] * ks16[None], axis=-1) * scale
    mm = jnp.max(s_mem, axis=-1, keepdims=True)
    m = jnp.maximum(mx, mm)
    e_mem = jnp.exp(s_mem - m)
    a = jnp.exp(mx - m)
    rd = 1.0 / (lx * a + jnp.sum(e_mem, axis=-1, keepdims=True))
    w_ref[0, :, :, :_TK] = ec * (a * rd)
    w_mem = e_mem * rd
    w_ref[0, :, :, _TK:] = w_mem
    wm16 = w_mem.astype(jnp.bfloat16).astype(jnp.float32)
    vs16 = vs.astype(jnp.bfloat16).astype(jnp.float32)
    o_ref[0] = oc * (a * rd) + jnp.sum(wm16[:, :, :, None] * vs16[None],
                                       axis=2)


def _merge_call(q, ksel, vsel, ec, oc, mx, lx):
    return pl.pallas_call(
        _merge_body,
        grid=(_B,),
        in_specs=[
            pl.BlockSpec((1, _H, _TQ, _DH), lambda b: (b, 0, 0, 0)),
            pl.BlockSpec((1, _TQ, _K, _DH), lambda b: (b, 0, 0, 0)),
            pl.BlockSpec((1, _TQ, _K, _DH), lambda b: (b, 0, 0, 0)),
            pl.BlockSpec((1, _H, _TQ, _TK), lambda b: (b, 0, 0, 0)),
            pl.BlockSpec((1, _H, _TQ, _DH), lambda b: (b, 0, 0, 0)),
            pl.BlockSpec((1, _H, _TQ, 1), lambda b: (b, 0, 0, 0)),
            pl.BlockSpec((1, _H, _TQ, 1), lambda b: (b, 0, 0, 0)),
        ],
        out_specs=[
            pl.BlockSpec((1, _H, _TQ, _DH), lambda b: (b, 0, 0, 0)),
            pl.BlockSpec((1, _H, _TQ, _TK + _K), lambda b: (b, 0, 0, 0)),
        ],
        out_shape=[
            jax.ShapeDtypeStruct((_B, _H, _TQ, _DH), jnp.float32),
            jax.ShapeDtypeStruct((_B, _H, _TQ, _TK + _K), jnp.float32),
        ],
    )(q, ksel, vsel, ec, oc, mx, lx)


def kernel(q, k_ctx, v_ctx, q_for_mem, mem_keys, mem_values):
    qf = q_for_mem.reshape(_NQ, _DH)
    keys, bmax = _score_call(qf, mem_keys)
    ksel, vsel = _topk_call(keys, bmax, mem_keys, mem_values)
    ksel = ksel.reshape(_B, _TQ, _K, _DH)
    vsel = vsel.reshape(_B, _TQ, _K, _DH)
    ec, oc, mx, lx = _ctx_call(q, k_ctx, v_ctx)
    return _merge_call(q, ksel, vsel, ec, oc, mx, lx)
